# Initial kernel scaffold; baseline (speedup 1.0000x reference)
import jax
import jax.numpy as jnp
from jax.experimental import pallas as pl


def kernel(xs, pairs, params):
    raise NotImplementedError("write your pallas kernel here")


# speedup vs baseline: 179.5599x; 179.5599x over previous
import jax
import jax.numpy as jnp
from jax import lax
from jax.experimental import pallas as pl

_BS, _NR, _NX = 8, 64, 32
_NN = _NR * _NX

_F32 = jnp.float32


def _mm(a, b):
    return jnp.dot(a, b, preferred_element_type=_F32)


def _d2v_body(pairs_ref,
              f1w, f1b, f2w, f2b, f3w, f3b, f4w, f4b, f5w, f5b,
              g1w, g1b, g2w, g2b,
              h1w, h1b, h2w, h2b, h3w, h3b, h4w, h4b, h5w, h5b,
              wa1w, wa1b, wa2w, wa2b, wb1w, wb1b, wb2w, wb2b,
              w0_out, w1_out):
    relu = lambda v: jnp.maximum(v, 0.0)
    x = pairs_ref[0]
    x = relu(_mm(x, f1w[...]) + f1b[...])
    x = x + relu(_mm(x, f2w[...]) + f2b[...])
    x = x + relu(_mm(x, f3w[...]) + f3b[...])
    x = x + relu(_mm(x, f4w[...]) + f4b[...])
    x = relu(_mm(x, f5w[...]) + f5b[...])
    x = jnp.mean(x.reshape(_NR, _NX, 64), axis=1)
    x = relu(_mm(x, g1w[...]) + g1b[...])
    x = relu(_mm(x, g2w[...]) + g2b[...])
    x = jnp.mean(x, axis=0, keepdims=True)
    x = relu(_mm(x, h1w[...]) + h1b[...])
    x = x + relu(_mm(x, h2w[...]) + h2b[...])
    x = x + relu(_mm(x, h3w[...]) + h3b[...])
    x = x + relu(_mm(x, h4w[...]) + h4b[...])
    d2v = relu(_mm(x, h5w[...]) + h5b[...])
    w0_out[0] = _mm(relu(_mm(d2v, wa1w[...]) + wa1b[...]),
                    wa2w[...]) + wa2b[...]
    w1_out[0] = _mm(relu(_mm(d2v, wb1w[...]) + wb1b[...]),
                    wb2w[...]) + wb2b[...]


_CR = 8
_CN = _CR * _NX
_NCH = _NR // _CR
_NEG = -1e30


def _softmax_rows(s):
    m = jnp.max(s, axis=-1, keepdims=True)
    e = jnp.exp(s - m)
    return e / (jnp.sum(e, axis=-1, keepdims=True) + 1e-16)


def _softmax_cols(s):
    m = jnp.max(s, axis=0, keepdims=True)
    e = jnp.exp(s - m)
    return e / (jnp.sum(e, axis=0, keepdims=True) + 1e-16)


def _gnn_body(xcol_ref, xrow_ref, w0r_ref, w0c_ref,
              as0_ref, ad0_ref, as0t_ref, ad0t_ref, b0_ref, b0t_ref,
              w1t_ref, w1tt_ref, as1t_ref, ad1_ref, b1_ref,
              wo_ref, bo_ref, out_ref):
    ii = lax.broadcasted_iota(jnp.int32, (_CN, _CN), 0) // _NX
    jj = lax.broadcasted_iota(jnp.int32, (_CN, _CN), 1) // _NX
    mask = ii == jj

    def masked(s):
        s = jnp.where(s >= 0.0, s, 0.2 * s)
        return jnp.where(mask, s, _NEG)

    xcol = xcol_ref[0, 0]
    xrow = xrow_ref[0, 0]
    h = xcol * w0r_ref[0]
    ht = w0c_ref[0] * xrow
    a_st = _mm(as0t_ref[0], ht)
    a_dt = _mm(ad0t_ref[0], ht)
    a_s = _mm(h, as0_ref[0])
    a_d = _mm(h, ad0_ref[0])

    parts, parts_t = [], []
    for g in range(2):
        hg = h[:, g * 32:(g + 1) * 32]
        hgt = ht[g * 32:(g + 1) * 32, :]
        s = masked(a_st[g:g + 1, :] + a_d[:, g:g + 1])
        parts.append(_mm(_softmax_rows(s), hg))
        st = masked(a_s[:, g:g + 1] + a_dt[g:g + 1, :])
        parts_t.append(_mm(hgt, _softmax_cols(st)))
    x2 = jnp.concatenate(parts, axis=-1) + b0_ref[0]
    x2t = jnp.concatenate(parts_t, axis=0) + b0t_ref[0]

    h2 = _mm(x2, w1t_ref[0])
    h2t = _mm(w1tt_ref[0], x2t)
    a_s2t = _mm(as1t_ref[0], h2t)
    a_d2 = _mm(h2, ad1_ref[0])
    parts = []
    for g in range(2):
        s = masked(a_s2t[g:g + 1, :] + a_d2[:, g:g + 1])
        parts.append(_mm(_softmax_rows(s), h2[:, g * 8:(g + 1) * 8]))
    x3 = jnp.concatenate(parts, axis=-1) + b1_ref[0]

    row = jnp.sum(x3.reshape(_CR, _NX, 16), axis=1)
    out_ref[0] = _mm(row, wo_ref[...]) + bo_ref[...]


def _full(shape):
    nd = len(shape)
    return pl.BlockSpec(shape, lambda b: (0,) * nd)


def _per_batch(shape):
    nd = len(shape)
    return pl.BlockSpec((1,) + shape, lambda b: (b,) + (0,) * nd)


def _full2(shape):
    nd = len(shape)
    return pl.BlockSpec(shape, lambda b, c: (0,) * nd)


def _per_b(shape):
    nd = len(shape)
    return pl.BlockSpec((1,) + shape, lambda b, c: (b,) + (0,) * nd)


def _per_bc(shape):
    nd = len(shape)
    return pl.BlockSpec((1, 1) + shape, lambda b, c: (b, c) + (0,) * nd)


@jax.jit
def kernel(xs, pairs, params):
    p = params

    def wt(name):
        W, b = p[name]
        return W.T.astype(_F32), b.reshape(1, -1).astype(_F32)

    names = ["f1", "f2r", "f3r", "f4r", "f5", "g1", "g2",
             "h1", "h2r", "h3r", "h4r", "h5",
             "wg0_1", "wg0_2", "wg1_1", "wg1_2"]
    wargs, wspecs = [], []
    for n in names:
        W, b = wt(n)
        wargs += [W, b]
        wspecs += [_full(W.shape), _full(b.shape)]

    pairs_f = pairs.reshape(_BS, _NN, 2)
    w0, w1 = pl.pallas_call(
        _d2v_body,
        grid=(_BS,),
        in_specs=[_per_batch((_NN, 2))] + wspecs,
        out_specs=[_per_batch((1, 320)), _per_batch((1, 1072))],
        out_shape=[jax.ShapeDtypeStruct((_BS, 1, 320), _F32),
                   jax.ShapeDtypeStruct((_BS, 1, 1072), _F32)],
    )(pairs_f, *wargs)
    w0 = w0.reshape(_BS, 320)
    w1 = w1.reshape(_BS, 1072)

    lin_w0 = w0[:, :128].reshape(_BS, 64, 2)
    w0row = lin_w0[:, :, 0].reshape(_BS, 1, 64)
    w0col = lin_w0[:, :, 0].reshape(_BS, 64, 1)
    a_src0 = w0[:, 128:192].reshape(_BS, 2, 32)
    a_dst0 = w0[:, 192:256].reshape(_BS, 2, 32)
    bias0 = w0[:, 256:320].reshape(_BS, 1, 64)
    bias0t = w0[:, 256:320].reshape(_BS, 64, 1)

    z32 = jnp.zeros((_BS, 32), _F32)
    A_src0 = jnp.stack(
        [jnp.concatenate([a_src0[:, 0, :], z32], axis=1),
         jnp.concatenate([z32, a_src0[:, 1, :]], axis=1)], axis=-1)
    A_dst0 = jnp.stack(
        [jnp.concatenate([a_dst0[:, 0, :], z32], axis=1),
         jnp.concatenate([z32, a_dst0[:, 1, :]], axis=1)], axis=-1)
    A_src0t = jnp.transpose(A_src0, (0, 2, 1))
    A_dst0t = jnp.transpose(A_dst0, (0, 2, 1))

    lin_w1 = w1[:, :1024].reshape(_BS, 16, 64)
    W1T = jnp.transpose(lin_w1, (0, 2, 1))
    a_src1 = w1[:, 1024:1040].reshape(_BS, 2, 8)
    a_dst1 = w1[:, 1040:1056].reshape(_BS, 2, 8)
    bias1 = w1[:, 1056:1072].reshape(_BS, 1, 16)

    z8 = jnp.zeros((_BS, 8), _F32)
    A_src1 = jnp.stack(
        [jnp.concatenate([a_src1[:, 0, :], z8], axis=1),
         jnp.concatenate([z8, a_src1[:, 1, :]], axis=1)], axis=-1)
    A_dst1 = jnp.stack(
        [jnp.concatenate([a_dst1[:, 0, :], z8], axis=1),
         jnp.concatenate([z8, a_dst1[:, 1, :]], axis=1)], axis=-1)
    A_src1t = jnp.transpose(A_src1, (0, 2, 1))

    Wo, bo = p["out_lin"]
    WoT = Wo.T.astype(_F32)
    bo = bo.reshape(1, 2).astype(_F32)

    xcol = xs.reshape(_BS, _NCH, _CN, 1).astype(_F32)
    xrow = xs.reshape(_BS, _NCH, 1, _CN).astype(_F32)

    out = pl.pallas_call(
        _gnn_body,
        grid=(_BS, _NCH),
        in_specs=[_per_bc((_CN, 1)), _per_bc((1, _CN)),
                  _per_b((1, 64)), _per_b((64, 1)),
                  _per_b((64, 2)), _per_b((64, 2)),
                  _per_b((2, 64)), _per_b((2, 64)),
                  _per_b((1, 64)), _per_b((64, 1)),
                  _per_b((64, 16)), _per_b((16, 64)),
                  _per_b((2, 16)), _per_b((16, 2)), _per_b((1, 16)),
                  _full2((16, 2)), _full2((1, 2))],
        out_specs=pl.BlockSpec((1, _CR, 2), lambda b, c: (b, c, 0)),
        out_shape=jax.ShapeDtypeStruct((_BS, _NR, 2), _F32),
    )(xcol, xrow, w0row, w0col,
      A_src0, A_dst0, A_src0t, A_dst0t, bias0, bias0t,
      W1T, lin_w1, A_src1t, A_dst1, bias1, WoT, bo)
    return out


# speedup vs baseline: 330.2334x; 1.8391x over previous
import jax
import jax.numpy as jnp
from jax import lax
from jax.experimental import pallas as pl

_BS, _NR, _NX = 8, 64, 32
_NN = _NR * _NX
_CR = 8
_CN = _CR * _NX
_GU = _NR // _CR
_NEG = -1e30

_F32 = jnp.float32


def _mm(a, b):
    return jnp.dot(a, b, preferred_element_type=_F32)


def _mm_t(w, x):
    return lax.dot_general(w, x, (((0,), (1,)), ((), ())),
                           preferred_element_type=_F32)


def _d2v_body(pairs_ref,
              f1w, f1b, f2w, f2b, f3w, f3b, f4w, f4b, f5w, f5b,
              g1w, g1b, g2w, g2b,
              h1w, h1b, h2w, h2b, h3w, h3b, h4w, h4b, h5w, h5b,
              wa1w, wa1b, wa2w, wa2b, wb1w, wb1b, wb2w, wb2b,
              w0_out, w1_out):
    relu = lambda v: jnp.maximum(v, 0.0)
    x = pairs_ref[...]
    x = relu(_mm(x, f1w[...]) + f1b[...])
    x = x + relu(_mm(x, f2w[...]) + f2b[...])
    x = x + relu(_mm(x, f3w[...]) + f3b[...])
    x = x + relu(_mm(x, f4w[...]) + f4b[...])
    x = relu(_mm(x, f5w[...]) + f5b[...])
    x = jnp.mean(x.reshape(_BS * _NR, _NX, 64), axis=1)
    x = relu(_mm(x, g1w[...]) + g1b[...])
    x = relu(_mm(x, g2w[...]) + g2b[...])
    x = jnp.mean(x.reshape(_BS, _NR, 64), axis=1)
    x = relu(_mm(x, h1w[...]) + h1b[...])
    x = x + relu(_mm(x, h2w[...]) + h2b[...])
    x = x + relu(_mm(x, h3w[...]) + h3b[...])
    x = x + relu(_mm(x, h4w[...]) + h4b[...])
    d2v = relu(_mm(x, h5w[...]) + h5b[...])
    w0_out[...] = _mm(relu(_mm(d2v, wa1w[...]) + wa1b[...]),
                      wa2w[...]) + wa2b[...]
    w1_out[...] = _mm(relu(_mm(d2v, wb1w[...]) + wb1b[...]),
                      wb2w[...]) + wb2b[...]


def _softmax_rows(s):
    m = jnp.max(s, axis=-1, keepdims=True)
    e = jnp.exp(s - m)
    return e / (jnp.sum(e, axis=-1, keepdims=True) + 1e-16)


def _attend(h, a_st, a_d, out, masked):
    s_parts = []
    for u in range(_GU):
        for g in range(2):
            row = a_st[g:g + 1, u * _CN:(u + 1) * _CN]
            col = a_d[u * _CN:(u + 1) * _CN, g:g + 1]
            s_parts.append(row + col)
    coef = _softmax_rows(masked(jnp.concatenate(s_parts, axis=0)))
    x_parts = []
    for u in range(_GU):
        aggs = [_mm(coef[(2 * u + g) * _CN:(2 * u + g + 1) * _CN, :],
                    h[u * _CN:(u + 1) * _CN, g * out:(g + 1) * out])
                for g in range(2)]
        x_parts.append(jnp.concatenate(aggs, axis=-1))
    return jnp.concatenate(x_parts, axis=0)


def _gnn_body(xcol_ref, w0r_ref,
              as0_ref, ad0_ref, b0_ref,
              w1t_ref, as1_ref, ad1_ref, b1_ref,
              wo_ref, bo_ref, out_ref):
    nw = _GU * 2 * _CN
    ii = (lax.broadcasted_iota(jnp.int32, (nw, _CN), 0) % _CN) // _NX
    jj = lax.broadcasted_iota(jnp.int32, (nw, _CN), 1) // _NX
    mask = ii == jj

    def masked(s):
        s = jnp.where(s >= 0.0, s, 0.2 * s)
        return jnp.where(mask, s, _NEG)

    xcol = xcol_ref[0]
    h = xcol * w0r_ref[0]
    a_st = _mm_t(as0_ref[0], h)
    a_d = _mm(h, ad0_ref[0])
    x2 = _attend(h, a_st, a_d, 32, masked) + b0_ref[0]

    h2 = _mm(x2, w1t_ref[0])
    a_s2t = _mm_t(as1_ref[0], h2)
    a_d2 = _mm(h2, ad1_ref[0])
    x3 = _attend(h2, a_s2t, a_d2, 8, masked) + b1_ref[0]

    row = jnp.sum(x3.reshape(_NR, _NX, 16), axis=1)
    out_ref[0] = _mm(row, wo_ref[...]) + bo_ref[...]


def _full(shape):
    nd = len(shape)
    return pl.BlockSpec(shape, lambda b: (0,) * nd)


def _per_batch(shape):
    nd = len(shape)
    return pl.BlockSpec((1,) + shape, lambda b: (b,) + (0,) * nd)


@jax.jit
def kernel(xs, pairs, params):
    p = params

    def wt(name):
        W, b = p[name]
        return W.T.astype(_F32), b.reshape(1, -1).astype(_F32)

    names = ["f1", "f2r", "f3r", "f4r", "f5", "g1", "g2",
             "h1", "h2r", "h3r", "h4r", "h5",
             "wg0_1", "wg0_2", "wg1_1", "wg1_2"]
    wargs = []
    for n in names:
        W, b = wt(n)
        wargs += [W, b]

    pairs_f = pairs.reshape(_BS * _NN, 2)
    w0, w1 = pl.pallas_call(
        _d2v_body,
        out_shape=[jax.ShapeDtypeStruct((_BS, 320), _F32),
                   jax.ShapeDtypeStruct((_BS, 1072), _F32)],
    )(pairs_f, *wargs)

    lin_w0 = w0[:, :128].reshape(_BS, 64, 2)
    w0row = lin_w0[:, :, 0].reshape(_BS, 1, 64)
    a_src0 = w0[:, 128:192].reshape(_BS, 2, 32)
    a_dst0 = w0[:, 192:256].reshape(_BS, 2, 32)
    bias0 = w0[:, 256:320].reshape(_BS, 1, 64)

    z32 = jnp.zeros((_BS, 32), _F32)
    A_src0 = jnp.stack(
        [jnp.concatenate([a_src0[:, 0, :], z32], axis=1),
         jnp.concatenate([z32, a_src0[:, 1, :]], axis=1)], axis=-1)
    A_dst0 = jnp.stack(
        [jnp.concatenate([a_dst0[:, 0, :], z32], axis=1),
         jnp.concatenate([z32, a_dst0[:, 1, :]], axis=1)], axis=-1)

    lin_w1 = w1[:, :1024].reshape(_BS, 16, 64)
    W1T = jnp.transpose(lin_w1, (0, 2, 1))
    a_src1 = w1[:, 1024:1040].reshape(_BS, 2, 8)
    a_dst1 = w1[:, 1040:1056].reshape(_BS, 2, 8)
    bias1 = w1[:, 1056:1072].reshape(_BS, 1, 16)

    z8 = jnp.zeros((_BS, 8), _F32)
    A_src1 = jnp.stack(
        [jnp.concatenate([a_src1[:, 0, :], z8], axis=1),
         jnp.concatenate([z8, a_src1[:, 1, :]], axis=1)], axis=-1)
    A_dst1 = jnp.stack(
        [jnp.concatenate([a_dst1[:, 0, :], z8], axis=1),
         jnp.concatenate([z8, a_dst1[:, 1, :]], axis=1)], axis=-1)

    Wo, bo = p["out_lin"]
    WoT = Wo.T.astype(_F32)
    bo = bo.reshape(1, 2).astype(_F32)

    xcol = xs.reshape(_BS, _NN, 1).astype(_F32)

    out = pl.pallas_call(
        _gnn_body,
        grid=(_BS,),
        in_specs=[_per_batch((_NN, 1)),
                  _per_batch((1, 64)),
                  _per_batch((64, 2)), _per_batch((64, 2)),
                  _per_batch((1, 64)),
                  _per_batch((64, 16)),
                  _per_batch((16, 2)), _per_batch((16, 2)),
                  _per_batch((1, 16)),
                  _full((16, 2)), _full((1, 2))],
        out_specs=_per_batch((_NR, 2)),
        out_shape=jax.ShapeDtypeStruct((_BS, _NR, 2), _F32),
    )(xcol, w0row,
      A_src0, A_dst0, bias0,
      W1T, A_src1, A_dst1, bias1, WoT, bo)
    return out
